# Initial kernel scaffold; baseline (speedup 1.0000x reference)
import jax
import jax.numpy as jnp
from jax.experimental import pallas as pl


def kernel(x, w):
    raise NotImplementedError("write your pallas kernel here")


# speedup vs baseline: 73.6240x; 73.6240x over previous
import functools

import jax
import jax.numpy as jnp
from jax import lax
from jax.experimental import pallas as pl
from jax.experimental.pallas import tpu as pltpu
from jax.experimental.pallas import tpu_sc as plsc

B = 1024
IN = 128
OUT = 64
K = 257
NSEG = 128
NC = 2
NS = 16
NW = NC * NS
IB = IN // NW
CHUNK = 256
NCHUNK = B // CHUNK


def _sc_body(tab_hbm, xt_hbm, out_hbm,
             tab_v, x_v, off_v, t_v, acc_v):
    c = lax.axis_index("c")
    s = lax.axis_index("s")
    wid = c * NS + s

    pltpu.sync_copy(xt_hbm.at[pl.ds(wid * IB, IB)], x_v)
    pltpu.sync_copy(tab_hbm.at[pl.ds(wid * (IB * K * OUT), IB * K * OUT)], tab_v)

    for i in range(IB):
        def pre(kk, carry, i=i):
            sl = pl.ds(kk * 16, 16)
            xx = x_v[i, sl]
            sid = ((xx + 1.0) * 64.0).astype(jnp.int32)
            sid = jnp.minimum(jnp.maximum(sid, 0), NSEG - 1)
            xmin = sid.astype(jnp.float32) * jnp.float32(2.0 / NSEG) - 1.0
            t = (xx - xmin) * jnp.float32(NSEG) - 1.0
            t_v[i, sl] = t
            off_v[i, sl] = (sid * 2 + i * K) * OUT
            return carry
        lax.fori_loop(0, B // 16, pre, None)

    GRP = CHUNK // 16
    for ch in range(NCHUNK):
        def body(g, carry, ch=ch):
            sl = pl.ds(g * 16, 16)
            offs = [off_v[i, sl] for i in range(IB)]
            ts = [t_v[i, sl] for i in range(IB)]
            f0s = [tv * (tv - 1.0) * 0.5 for tv in ts]
            f1s = [1.0 - tv * tv for tv in ts]
            f2s = [tv * (tv + 1.0) * 0.5 for tv in ts]
            gl = g - ch * GRP
            for lane in range(16):
                acc = [jnp.zeros((16,), jnp.float32) for _ in range(4)]
                for i in range(IB):
                    off = offs[i][lane]
                    for j, fj in ((0, f0s[i][lane]),
                                  (1, f1s[i][lane]),
                                  (2, f2s[i][lane])):
                        base = off + j * OUT
                        for k in range(4):
                            row = tab_v[pl.ds(base + k * 16, 16)]
                            acc[k] = acc[k] + fj * row
                bl = gl * 16 + lane
                for k in range(4):
                    acc_v[bl, pl.ds(k * 16, 16)] = acc[k]
            return carry
        lax.fori_loop(ch * GRP, (ch + 1) * GRP, body, None)
        pltpu.sync_copy(acc_v, out_hbm.at[wid, pl.ds(ch * CHUNK, CHUNK)])


@functools.partial(
    pl.kernel,
    out_type=jax.ShapeDtypeStruct((NW, B, OUT), jnp.float32),
    mesh=plsc.VectorSubcoreMesh(core_axis_name="c", subcore_axis_name="s"),
    scratch_types=[
        pltpu.VMEM((IB * K * OUT,), jnp.float32),
        pltpu.VMEM((IB, B), jnp.float32),
        pltpu.VMEM((IB, B), jnp.int32),
        pltpu.VMEM((IB, B), jnp.float32),
        pltpu.VMEM((CHUNK, OUT), jnp.float32),
    ],
)
def _piecewise_sc(tab_hbm, xt_hbm, out_hbm, *scratch):
    _sc_body(tab_hbm, xt_hbm, out_hbm, *scratch)


def _add_body(p_ref, o_ref):
    o_ref[...] = jnp.sum(p_ref[...], axis=0)


_add_parts = pl.pallas_call(
    _add_body,
    out_shape=jax.ShapeDtypeStruct((B, OUT), jnp.float32),
)


def kernel(x, w):
    xt = x.T
    tab = jnp.transpose(w, (1, 2, 0)).reshape(-1)
    parts = _piecewise_sc(tab, xt)
    return _add_parts(parts)
